# baseline (speedup 1.0000x reference)
import functools

import jax
import jax.numpy as jnp
from jax import lax
from jax.experimental import pallas as pl
from jax.experimental.pallas import tpu as pltpu

CODEBOOK = 1024
DIM = 256
N_TOKENS = 16 * 576
BLK = 512
HBLK = BLK // 2


def _half(x, emb2, esqb_ref, iota_k):
    dott2 = lax.dot_general(emb2, x, (((1,), (1,)), ((), ())),
                            preferred_element_type=jnp.float32)
    dist = dott2 - esqb_ref[...]
    m = jnp.max(dist, axis=0)
    cand = jnp.where(dist == m[None, :], iota_k, CODEBOOK)
    idx = jnp.min(cand, axis=0).astype(jnp.int32)
    onehot = (iota_k == idx[None, :]).astype(jnp.float32)
    return idx, onehot


def _vq_kernel(x_ref, embed_ref, q_ref, idx_ref, perp_ref,
               esqb_ref, cacc_ref, emb2_ref):
    i = pl.program_id(0)
    nsteps = pl.num_programs(0)

    x = x_ref[...]
    emb = embed_ref[...]

    @pl.when(i == 0)
    def _prep():
        emb_sq = jnp.sum(emb * emb, axis=1)
        esqb_ref[...] = jnp.broadcast_to(emb_sq[:, None], (CODEBOOK, HBLK))
        cacc_ref[...] = jnp.zeros_like(cacc_ref)
        emb2_ref[...] = emb + emb

    iota_k = lax.broadcasted_iota(jnp.int32, (CODEBOOK, HBLK), 0)
    emb2 = emb2_ref[...]

    idx_a, onehot_a = _half(x[:HBLK, :], emb2, esqb_ref, iota_k)
    idx_b, onehot_b = _half(x[HBLK:, :], emb2, esqb_ref, iota_k)

    idx_ref[...] = jnp.concatenate([idx_a, idx_b]).reshape(1, 1, BLK)

    cacc_ref[...] += onehot_a + onehot_b

    q_ref[:HBLK, :] = lax.dot_general(onehot_a, emb, (((0,), (0,)), ((), ())),
                                      preferred_element_type=jnp.float32)
    q_ref[HBLK:, :] = lax.dot_general(onehot_b, emb, (((0,), (0,)), ((), ())),
                                      preferred_element_type=jnp.float32)

    @pl.when(i == nsteps - 1)
    def _fin():
        counts = jnp.sum(cacc_ref[...], axis=1)
        probs = counts / float(N_TOKENS)
        ent = jnp.sum(probs * jnp.log(probs + 1e-10), keepdims=True)
        perp_ref[...] = jnp.exp(-ent).reshape(1, 1)


@jax.jit
def kernel(x, embed):
    shape = x.shape
    flat = x.reshape(-1, DIM)
    grid = N_TOKENS // BLK

    q, idx3, perp = pl.pallas_call(
        _vq_kernel,
        grid=(grid,),
        in_specs=[
            pl.BlockSpec((BLK, DIM), lambda i: (i, 0)),
            pl.BlockSpec((CODEBOOK, DIM), lambda i: (0, 0)),
        ],
        out_specs=[
            pl.BlockSpec((BLK, DIM), lambda i: (i, 0)),
            pl.BlockSpec((1, 1, BLK), lambda i: (i, 0, 0)),
            pl.BlockSpec((1, 1), lambda i: (0, 0)),
        ],
        out_shape=[
            jax.ShapeDtypeStruct((N_TOKENS, DIM), jnp.float32),
            jax.ShapeDtypeStruct((grid, 1, BLK), jnp.int32),
            jax.ShapeDtypeStruct((1, 1), jnp.float32),
        ],
        scratch_shapes=[
            pltpu.VMEM((CODEBOOK, HBLK), jnp.float32),
            pltpu.VMEM((CODEBOOK, HBLK), jnp.float32),
            pltpu.VMEM((CODEBOOK, DIM), jnp.float32),
        ],
    )(flat, embed)

    quantize = q.reshape(shape)
    embed_ind = idx3.reshape(shape[:-1])
    perplexity = perp.reshape(())
    return quantize, embed_ind, perplexity


# speedup vs baseline: 1.1246x; 1.1246x over previous
import functools

import jax
import jax.numpy as jnp
from jax import lax
from jax.experimental import pallas as pl
from jax.experimental.pallas import tpu as pltpu

CODEBOOK = 1024
DIM = 256
N_TOKENS = 16 * 576
BLK = 768
HBLK = BLK // 3


def _half(x, emb2, esqb_ref, iota_k):
    dott2 = lax.dot_general(emb2, x, (((1,), (1,)), ((), ())),
                            preferred_element_type=jnp.float32)
    dist = dott2 - esqb_ref[...]
    m = jnp.max(dist, axis=0)
    cand = jnp.where(dist == m[None, :], iota_k, CODEBOOK)
    idx = jnp.min(cand, axis=0).astype(jnp.int32)
    onehot = (iota_k == idx[None, :]).astype(jnp.float32)
    return idx, onehot


def _vq_kernel(x_ref, embed_ref, q_ref, idx_ref, perp_ref,
               esqb_ref, cacc_ref, emb2_ref):
    i = pl.program_id(0)
    nsteps = pl.num_programs(0)

    x = x_ref[...]
    emb = embed_ref[...]

    @pl.when(i == 0)
    def _prep():
        emb_sq = jnp.sum(emb * emb, axis=1)
        esqb_ref[...] = jnp.broadcast_to(emb_sq[:, None], (CODEBOOK, HBLK))
        cacc_ref[...] = jnp.zeros_like(cacc_ref)
        emb2_ref[...] = emb + emb

    iota_k = lax.broadcasted_iota(jnp.int32, (CODEBOOK, HBLK), 0)
    emb2 = emb2_ref[...]

    idx_a, onehot_a = _half(x[:HBLK, :], emb2, esqb_ref, iota_k)
    idx_b, onehot_b = _half(x[HBLK:2 * HBLK, :], emb2, esqb_ref, iota_k)
    idx_c, onehot_c = _half(x[2 * HBLK:, :], emb2, esqb_ref, iota_k)

    idx_ref[...] = jnp.concatenate([idx_a, idx_b, idx_c]).reshape(1, 1, BLK)

    cacc_ref[...] += onehot_a + onehot_b + onehot_c

    q_ref[:HBLK, :] = lax.dot_general(onehot_a, emb, (((0,), (0,)), ((), ())),
                                      preferred_element_type=jnp.float32)
    q_ref[HBLK:2 * HBLK, :] = lax.dot_general(onehot_b, emb, (((0,), (0,)), ((), ())),
                                              preferred_element_type=jnp.float32)
    q_ref[2 * HBLK:, :] = lax.dot_general(onehot_c, emb, (((0,), (0,)), ((), ())),
                                          preferred_element_type=jnp.float32)

    @pl.when(i == nsteps - 1)
    def _fin():
        counts = jnp.sum(cacc_ref[...], axis=1)
        probs = counts / float(N_TOKENS)
        ent = jnp.sum(probs * jnp.log(probs + 1e-10), keepdims=True)
        perp_ref[...] = jnp.exp(-ent).reshape(1, 1)


@jax.jit
def kernel(x, embed):
    shape = x.shape
    flat = x.reshape(-1, DIM)
    grid = N_TOKENS // BLK

    q, idx3, perp = pl.pallas_call(
        _vq_kernel,
        grid=(grid,),
        in_specs=[
            pl.BlockSpec((BLK, DIM), lambda i: (i, 0)),
            pl.BlockSpec((CODEBOOK, DIM), lambda i: (0, 0)),
        ],
        out_specs=[
            pl.BlockSpec((BLK, DIM), lambda i: (i, 0)),
            pl.BlockSpec((1, 1, BLK), lambda i: (i, 0, 0)),
            pl.BlockSpec((1, 1), lambda i: (0, 0)),
        ],
        out_shape=[
            jax.ShapeDtypeStruct((N_TOKENS, DIM), jnp.float32),
            jax.ShapeDtypeStruct((grid, 1, BLK), jnp.int32),
            jax.ShapeDtypeStruct((1, 1), jnp.float32),
        ],
        scratch_shapes=[
            pltpu.VMEM((CODEBOOK, HBLK), jnp.float32),
            pltpu.VMEM((CODEBOOK, HBLK), jnp.float32),
            pltpu.VMEM((CODEBOOK, DIM), jnp.float32),
        ],
    )(flat, embed)

    quantize = q.reshape(shape)
    embed_ind = idx3.reshape(shape[:-1])
    perplexity = perp.reshape(())
    return quantize, embed_ind, perplexity
